# Initial kernel scaffold; baseline (speedup 1.0000x reference)
import jax
import jax.numpy as jnp
from jax.experimental import pallas as pl


def kernel(edge_feats, node_feats, edge_index, W_e, b_e, W_n, b_n, W_g1, b_g1, W_g2, b_g2, W_o1, b_o1, W_o2, b_o2):
    raise NotImplementedError("write your pallas kernel here")


# speedup vs baseline: 4.9626x; 4.9626x over previous
import functools

import jax
import jax.numpy as jnp
from jax import lax
from jax.experimental import pallas as pl
from jax.experimental.pallas import tpu as pltpu
from jax.experimental.pallas import tpu_sc as plsc

N_NODES = 10000
N_EDGES = 320000
HIDDEN = 128

NC = 2
NS = 16
NW = NC * NS
LANES = 16

CHUNK = 80
EPT = N_EDGES // NW
NCH = EPT // CHUNK
EPT_DEG = N_EDGES // NS
NCH_DEG = EPT_DEG // CHUNK
N_PAD = 10240
ROWS_PT = N_PAD // NS
DEG_W = 16

_MESH = plsc.VectorSubcoreMesh(core_axis_name="c", subcore_axis_name="s")
_f32 = jnp.float32


@functools.partial(
    pl.kernel,
    out_type=jax.ShapeDtypeStruct((NC, N_PAD, DEG_W), _f32),
    mesh=_MESH,
    scratch_types=[
        pltpu.VMEM((NCH_DEG, CHUNK), jnp.int32),
        pltpu.VMEM((CHUNK, DEG_W), _f32),
        pltpu.VMEM_SHARED((N_PAD, DEG_W), _f32),
    ],
    compiler_params=pltpu.CompilerParams(use_tc_tiling_on_sc=False),
)
def _sc_degrees(ei, z16, ones, out, idx_v, ones_v, acc):
    c = lax.axis_index("c")
    s = lax.axis_index("s")
    pltpu.sync_copy(ei.at[c, s], idx_v)
    pltpu.sync_copy(ones, ones_v)
    pltpu.sync_copy(z16, acc.at[pl.ds(s * ROWS_PT, ROWS_PT)])
    plsc.subcore_barrier()

    @pl.loop(0, NCH_DEG)
    def _(j):
        pltpu.sync_copy(ones_v, acc.at[idx_v.at[j]], add=True)

    plsc.subcore_barrier()
    pltpu.sync_copy(acc.at[pl.ds(s * ROWS_PT, ROWS_PT)],
                    out.at[c, pl.ds(s * ROWS_PT, ROWS_PT)])


@functools.partial(
    pl.kernel,
    out_type=jax.ShapeDtypeStruct((NC, N_PAD, HIDDEN), _f32),
    mesh=_MESH,
    scratch_types=[
        pltpu.VMEM((NCH, CHUNK), jnp.int32),
        pltpu.VMEM((NCH, CHUNK), jnp.int32),
        pltpu.VMEM((CHUNK, HIDDEN), _f32),
        pltpu.VMEM_SHARED((N_PAD, HIDDEN), _f32),
        pltpu.SemaphoreType.DMA,
    ],
)
def _sc_aggregate(g, ei, z, out, idx_s, idx_d, rows, acc, sem):
    c = lax.axis_index("c")
    s = lax.axis_index("s")
    w = c * NS + s
    pltpu.sync_copy(ei.at[0, w], idx_s)
    pltpu.sync_copy(ei.at[1, w], idx_d)
    pltpu.sync_copy(z, acc.at[pl.ds(s * ROWS_PT, ROWS_PT)])
    plsc.subcore_barrier()

    @pl.loop(0, NCH)
    def _(j):
        pltpu.async_copy(g.at[idx_s.at[j]], rows, sem).wait()
        pltpu.sync_copy(rows, acc.at[idx_d.at[j]], add=True)

    plsc.subcore_barrier()
    pltpu.sync_copy(acc.at[pl.ds(s * ROWS_PT, ROWS_PT)],
                    out.at[c, pl.ds(s * ROWS_PT, ROWS_PT)])


@functools.partial(
    pl.kernel,
    out_type=jax.ShapeDtypeStruct((N_EDGES,), _f32),
    mesh=_MESH,
    scratch_types=[
        pltpu.VMEM((NCH, CHUNK), jnp.int32),
        pltpu.VMEM((NCH, CHUNK), jnp.int32),
        pltpu.VMEM((CHUNK, HIDDEN), _f32),
        pltpu.VMEM((CHUNK, HIDDEN), _f32),
        pltpu.VMEM((CHUNK,), _f32),
        pltpu.VMEM((HIDDEN,), _f32),
        pltpu.VMEM((HIDDEN,), _f32),
        pltpu.VMEM((LANES,), _f32),
        pltpu.SemaphoreType.DMA,
    ],
)
def _sc_edge(p, ei, b1, w2, b2, out,
             idx_s, idx_d, buf_s, buf_d, res, b1_v, w2_v, b2_v, sem):
    c = lax.axis_index("c")
    s = lax.axis_index("s")
    w = c * NS + s
    base = w * EPT
    pltpu.sync_copy(ei.at[0, w], idx_s)
    pltpu.sync_copy(ei.at[1, w], idx_d)
    pltpu.sync_copy(b1, b1_v)
    pltpu.sync_copy(w2, w2_v)
    pltpu.sync_copy(b2, b2_v)
    lane = lax.iota(jnp.int32, LANES)
    perms = [jnp.bitwise_xor(lane, sh) for sh in (8, 4, 2, 1)]
    b1s = [b1_v[pl.ds(q * LANES, LANES)] for q in range(HIDDEN // LANES)]
    w2s = [w2_v[pl.ds(q * LANES, LANES)] for q in range(HIDDEN // LANES)]
    b2vec = b2_v[...]

    _dnums = lax.GatherDimensionNumbers(
        offset_dims=(), collapsed_slice_dims=(0,), start_index_map=(0,))

    def hsum(v):
        for perm in perms:
            shuf = lax.gather(v, perm[:, None], _dnums, slice_sizes=(1,),
                              mode=lax.GatherScatterMode.PROMISE_IN_BOUNDS)
            v = v + shuf
        return v

    @pl.loop(0, NCH)
    def _(j):
        pltpu.async_copy(p.at[idx_s.at[j]], buf_s, sem).wait()
        pltpu.async_copy(p.at[idx_d.at[j]], buf_d, sem).wait()

        @pl.loop(0, CHUNK // LANES)
        def _(gi):
            vout = jnp.zeros((LANES,), _f32)
            for l in range(LANES):
                e = gi * LANES + l
                acc = jnp.zeros((LANES,), _f32)
                for q in range(HIDDEN // LANES):
                    sq = buf_s[e, pl.ds(q * LANES, LANES)]
                    dq = buf_d[e, pl.ds(q * LANES, LANES)]
                    t = jnp.maximum(sq + dq + b1s[q], 0.0)
                    acc = acc + t * w2s[q]
                vout = jnp.where(lane == l, hsum(acc), vout)
            res[pl.ds(gi * LANES, LANES)] = vout + b2vec

        pltpu.sync_copy(res, out.at[pl.ds(base + j * CHUNK, CHUNK)])


def _tc_embed_body(deg_ref, nf_ref, wn_ref, bn_ref, g1_ref, nin_ref, nout_ref):
    deg = deg_ref[...]
    nout = lax.rsqrt(jnp.clip(deg[0][:N_NODES, 0:1], 1.0, None))
    nin = lax.rsqrt(jnp.clip(deg[1][:N_NODES, 0:1], 1.0, None))
    h0 = jnp.dot(nf_ref[...], wn_ref[...], preferred_element_type=_f32, precision=lax.Precision.HIGHEST)
    h0 = h0 + bn_ref[...]
    g1_ref[...] = h0 * nout
    nin_ref[...] = nin
    nout_ref[...] = nout


def _tc_embed(degs, nf, Wn, bn):
    return pl.pallas_call(
        _tc_embed_body,
        out_shape=(
            jax.ShapeDtypeStruct((N_NODES, HIDDEN), _f32),
            jax.ShapeDtypeStruct((N_NODES, 1), _f32),
            jax.ShapeDtypeStruct((N_NODES, 1), _f32),
        ),
    )(degs, nf, Wn, bn)


def _tc_layer_body(parts_ref, nin_ref, nout_ref, w_ref, b_ref, out_ref):
    parts = parts_ref[...]
    agg = (parts[0, :N_NODES] + parts[1, :N_NODES]) * nin_ref[...]
    h = jnp.dot(agg, w_ref[...], preferred_element_type=_f32, precision=lax.Precision.HIGHEST) + b_ref[...]
    out_ref[...] = jnp.maximum(h, 0.0) * nout_ref[...]


def _tc_layer(parts, nin, nout, W, b):
    return pl.pallas_call(
        _tc_layer_body,
        out_shape=jax.ShapeDtypeStruct((N_NODES, HIDDEN), _f32),
    )(parts, nin, nout, W, b)


def _tc_final_body(parts_ref, nin_ref, w_ref, b_ref, wo1_ref, out_ref):
    parts = parts_ref[...]
    agg = (parts[0, :N_NODES] + parts[1, :N_NODES]) * nin_ref[...]
    h = jnp.dot(agg, w_ref[...], preferred_element_type=_f32, precision=lax.Precision.HIGHEST) + b_ref[...]
    h = jnp.maximum(h, 0.0)
    out_ref[...] = jnp.dot(h, wo1_ref[...], preferred_element_type=_f32, precision=lax.Precision.HIGHEST)


def _tc_final(parts, nin, W, b, Wo1):
    return pl.pallas_call(
        _tc_final_body,
        out_shape=jax.ShapeDtypeStruct((N_NODES, HIDDEN), _f32),
    )(parts, nin, W, b, Wo1)


def kernel(edge_feats, node_feats, edge_index, W_e, b_e, W_n, b_n,
           W_g1, b_g1, W_g2, b_g2, W_o1, b_o1, W_o2, b_o2):
    del edge_feats, W_e, b_e
    ei_deg = edge_index.reshape(2, NS, NCH_DEG, CHUNK)
    ei_lay = edge_index.reshape(2, NW, NCH, CHUNK)
    z16 = jnp.zeros((ROWS_PT, DEG_W), _f32)
    z128 = jnp.zeros((ROWS_PT, HIDDEN), _f32)
    ones16 = jnp.ones((CHUNK, DEG_W), _f32)

    degs = _sc_degrees(ei_deg, z16, ones16)
    g1, nin, nout = _tc_embed(degs, node_feats, W_n, b_n.reshape(1, -1))
    parts1 = _sc_aggregate(g1, ei_lay, z128)
    g2 = _tc_layer(parts1, nin, nout, W_g1, b_g1.reshape(1, -1))
    parts2 = _sc_aggregate(g2, ei_lay, z128)
    p = _tc_final(parts2, nin, W_g2, b_g2.reshape(1, -1), W_o1)

    b2v = jnp.full((LANES,), b_o2[0], _f32)
    preds = _sc_edge(p, ei_lay, b_o1, W_o2.reshape(-1), b2v)
    return preds[:, None]
